# baseline (device time: 20998 ns/iter reference)
import jax
import jax.numpy as jnp
from jax import lax
from jax.experimental import pallas as pl
from jax.experimental.pallas import tpu as pltpu

Z = 4
CHUNKS = 4


def kernel(Q, K, V):
    b, q, h, d = Q.shape
    kseq = K.shape[1]
    kc = kseq // CHUNKS
    scale = d ** -0.5

    Kt = jnp.transpose(K, (0, 2, 3, 1))
    Vt = jnp.transpose(V, (0, 2, 3, 1))
    Qt = jnp.transpose(Q * scale, (0, 2, 3, 1))

    def body(
        q_ref, k_hbm, v_hbm, out_ref,
        k_buf, v_buf, send_buf, comm_ref,
        k_sems, v_sems, send_sems, recv_sems,
    ):
        my_x = lax.axis_index("x")
        my_y = lax.axis_index("y")
        my_z = lax.axis_index("z")

        barrier_sem = pltpu.get_barrier_semaphore()
        for off in range(1, Z):
            pl.semaphore_signal(
                barrier_sem,
                inc=1,
                device_id=(my_x, my_y, (my_z + off) % Z),
                device_id_type=pl.DeviceIdType.MESH,
            )

        def start_load(c):
            slot = c % 2
            kcp = pltpu.make_async_copy(
                k_hbm.at[:, :, :, pl.ds(c * kc, kc)], k_buf.at[slot], k_sems.at[slot]
            )
            vcp = pltpu.make_async_copy(
                v_hbm.at[:, :, :, pl.ds(c * kc, kc)], v_buf.at[slot], v_sems.at[slot]
            )
            kcp.start()
            vcp.start()
            return kcp, vcp

        copies = {0: start_load(0)}
        qt = q_ref[:]
        o_acc = jnp.zeros((b, h, d), jnp.float32)
        l_acc = jnp.zeros((b, h), jnp.float32)
        for c in range(CHUNKS):
            if c + 1 < CHUNKS:
                copies[c + 1] = start_load(c + 1)
            kcp, vcp = copies[c]
            kcp.wait()
            vcp.wait()
            slot = c % 2
            p = jnp.exp(jnp.sum(qt * k_buf[slot], axis=2, keepdims=True))
            l_acc = l_acc + jnp.sum(p, axis=3)[:, :, 0]
            o_acc = o_acc + jnp.sum(p * v_buf[slot], axis=3)

        l_row = jnp.pad(l_acc[:, None, :], ((0, 0), (0, 0), (0, d - h)))
        send_buf[:] = jnp.concatenate([o_acc, l_row], axis=1)

        pl.semaphore_wait(barrier_sem, Z - 1)

        rdmas = []
        for off in range(1, Z):
            rdma = pltpu.make_async_remote_copy(
                src_ref=send_buf,
                dst_ref=comm_ref.at[off - 1],
                send_sem=send_sems.at[off - 1],
                recv_sem=recv_sems.at[off - 1],
                device_id=(my_x, my_y, (my_z + off) % Z),
                device_id_type=pl.DeviceIdType.MESH,
            )
            rdma.start()
            rdmas.append(rdma)

        for rdma in rdmas:
            rdma.wait_recv()

        total = send_buf[:] + comm_ref[0] + comm_ref[1] + comm_ref[2]
        o_sum = total[:, :h, :]
        l_sum = total[:, h, :h]
        out_ref[:] = (o_sum / l_sum[:, :, None])[:, None, :, :]

        for rdma in rdmas:
            rdma.wait_send()

    out_shape = jax.ShapeDtypeStruct((b, q, h, d), jnp.float32)
    return pl.pallas_call(
        body,
        out_shape=out_shape,
        in_specs=[
            pl.BlockSpec(memory_space=pltpu.VMEM),
            pl.BlockSpec(memory_space=pl.ANY),
            pl.BlockSpec(memory_space=pl.ANY),
        ],
        out_specs=pl.BlockSpec(memory_space=pltpu.VMEM),
        scratch_shapes=[
            pltpu.VMEM((2, b, h, d, kc), jnp.float32),
            pltpu.VMEM((2, b, h, d, kc), jnp.float32),
            pltpu.VMEM((b, h + 1, d), jnp.float32),
            pltpu.VMEM((Z - 1, b, h + 1, d), jnp.float32),
            pltpu.SemaphoreType.DMA((2,)),
            pltpu.SemaphoreType.DMA((2,)),
            pltpu.SemaphoreType.DMA((Z - 1,)),
            pltpu.SemaphoreType.DMA((Z - 1,)),
        ],
        compiler_params=pltpu.CompilerParams(collective_id=0),
    )(Qt, Kt, Vt)


# device time: 18084 ns/iter; 1.1611x vs baseline; 1.1611x over previous
import jax
import jax.numpy as jnp
from jax import lax
from jax.experimental import pallas as pl
from jax.experimental.pallas import tpu as pltpu

Z = 4
CHUNKS = 4


def kernel(Q, K, V):
    b, q, h, d = Q.shape
    kseq = K.shape[1]
    bc = b // CHUNKS
    scale = d ** -0.5

    Kt = jnp.transpose(K, (0, 2, 3, 1))
    Vt = jnp.transpose(V, (0, 2, 3, 1))
    Qt = jnp.transpose(Q * scale, (0, 2, 3, 1))

    def body(
        q_ref, k_hbm, v_hbm, out_ref,
        k_buf, v_buf, send_buf, comm_ref,
        k_sems, v_sems, send_sems, recv_sems,
    ):
        my_x = lax.axis_index("x")
        my_y = lax.axis_index("y")
        my_z = lax.axis_index("z")

        barrier_sem = pltpu.get_barrier_semaphore()
        for off in range(1, Z):
            pl.semaphore_signal(
                barrier_sem,
                inc=1,
                device_id=(my_x, my_y, (my_z + off) % Z),
                device_id_type=pl.DeviceIdType.MESH,
            )

        def start_load(c):
            slot = c % 2
            kcp = pltpu.make_async_copy(
                k_hbm.at[pl.ds(c * bc, bc)], k_buf.at[slot], k_sems.at[slot]
            )
            vcp = pltpu.make_async_copy(
                v_hbm.at[pl.ds(c * bc, bc)], v_buf.at[slot], v_sems.at[slot]
            )
            kcp.start()
            vcp.start()
            return kcp, vcp

        copies = {0: start_load(0)}
        qt = q_ref[:]
        for c in range(CHUNKS):
            if c + 1 < CHUNKS:
                copies[c + 1] = start_load(c + 1)
            kcp, vcp = copies[c]
            kcp.wait()
            vcp.wait()
            slot = c % 2
            qc = qt[c * bc:(c + 1) * bc]
            p = jnp.exp(jnp.sum(qc * k_buf[slot], axis=2, keepdims=True))
            l_c = jnp.sum(p, axis=3)[:, :, 0]
            o_c = jnp.sum(p * v_buf[slot], axis=3)
            l_row = jnp.pad(l_c[:, None, :], ((0, 0), (0, 0), (0, d - h)))
            send_buf[pl.ds(c * bc, bc)] = jnp.concatenate([o_c, l_row], axis=1)

        pl.semaphore_wait(barrier_sem, Z - 1)

        rdmas = []
        for off in range(1, Z):
            rdma = pltpu.make_async_remote_copy(
                src_ref=send_buf,
                dst_ref=comm_ref.at[off - 1],
                send_sem=send_sems.at[off - 1],
                recv_sem=recv_sems.at[off - 1],
                device_id=(my_x, my_y, (my_z + off) % Z),
                device_id_type=pl.DeviceIdType.MESH,
            )
            rdma.start()
            rdmas.append(rdma)

        for rdma in rdmas:
            rdma.wait_recv()

        total = send_buf[:] + comm_ref[0] + comm_ref[1] + comm_ref[2]
        o_sum = total[:, :h, :]
        l_sum = total[:, h, :h]
        out_ref[:] = (o_sum / l_sum[:, :, None])[:, None, :, :]

        for rdma in rdmas:
            rdma.wait_send()

    out_shape = jax.ShapeDtypeStruct((b, q, h, d), jnp.float32)
    return pl.pallas_call(
        body,
        out_shape=out_shape,
        in_specs=[
            pl.BlockSpec(memory_space=pltpu.VMEM),
            pl.BlockSpec(memory_space=pl.ANY),
            pl.BlockSpec(memory_space=pl.ANY),
        ],
        out_specs=pl.BlockSpec(memory_space=pltpu.VMEM),
        scratch_shapes=[
            pltpu.VMEM((2, bc, h, d, kseq), jnp.float32),
            pltpu.VMEM((2, bc, h, d, kseq), jnp.float32),
            pltpu.VMEM((b, h + 1, d), jnp.float32),
            pltpu.VMEM((Z - 1, b, h + 1, d), jnp.float32),
            pltpu.SemaphoreType.DMA((2,)),
            pltpu.SemaphoreType.DMA((2,)),
            pltpu.SemaphoreType.DMA((Z - 1,)),
            pltpu.SemaphoreType.DMA((Z - 1,)),
        ],
        compiler_params=pltpu.CompilerParams(collective_id=0),
    )(Qt, Kt, Vt)
